# baseline (speedup 1.0000x reference)
import functools

import jax
import jax.numpy as jnp
from jax import lax
from jax.experimental import pallas as pl
from jax.experimental.pallas import tpu as pltpu
from jax.experimental.pallas import tpu_sc as plsc


def _chunks_of_L(L):
    chunks = []
    off = 0
    while off < L:
        size = min(128, L - off)
        if L - off > 128:
            size -= size % 8
        chunks.append((off, size))
        off += size
    return chunks


def _make_sc_sum(B, L, V2, D, NW):
    assert B % NW == 0
    dpw = B // NW
    assert (dpw * L) % 8 == 0 and (L % 8) == 0
    chunks = _chunks_of_L(L)
    n_groups = D // 16
    BLK = 16
    n_blk, tail = divmod(L, BLK)
    mesh = plsc.VectorSubcoreMesh(core_axis_name="c", subcore_axis_name="s")
    NC = mesh.num_cores

    @functools.partial(
        pl.kernel,
        out_type=jax.ShapeDtypeStruct((B, D), jnp.float32),
        mesh=mesh,
        compiler_params=pltpu.CompilerParams(use_tc_tiling_on_sc=True),
        scratch_types=[
            pltpu.VMEM((dpw * L + BLK,), jnp.int32),
            pltpu.VMEM((dpw * L,), jnp.int32),
            pltpu.VMEM((L, 2 * D), jnp.float32),
            pltpu.VMEM((L, 2 * D), jnp.float32),
            pltpu.VMEM((dpw, D), jnp.float32),
            pltpu.SemaphoreType.DMA,
            pltpu.SemaphoreType.DMA,
        ],
    )
    def sc_sum(tokens_hbm, idx2_hbm, table_hbm, out_hbm,
               tok_v, idx2_v, buf_a, buf_b, outblk, sem_a, sem_b):
        wid = lax.axis_index("s") * NC + lax.axis_index("c")
        base_doc = wid * dpw

        pltpu.sync_copy(tokens_hbm.at[pl.ds(base_doc * L, dpw * L)],
                        tok_v.at[pl.ds(0, dpw * L)])
        pltpu.sync_copy(idx2_hbm.at[pl.ds(base_doc * L, dpw * L)], idx2_v)

        def gather_start(d, buf, sem):
            off = d * L
            for c_off, c_sz in chunks:
                pltpu.async_copy(
                    table_hbm.at[idx2_v.at[pl.ds(off + c_off, c_sz)]],
                    buf.at[pl.ds(c_off, c_sz)],
                    sem,
                )

        def gather_wait(buf, sem):
            for c_off, c_sz in chunks:
                pltpu.make_async_copy(
                    table_hbm.at[idx2_v.at[pl.ds(c_off, c_sz)]],
                    buf.at[pl.ds(c_off, c_sz)],
                    sem,
                ).wait()

        def reduce_rows(buf, off, t0, n_rows, accs):
            par = (tok_v[pl.ds(off + t0, BLK)] & 1) * D
            for j in range(n_rows):
                coff = par[j]
                for g in range(n_groups):
                    accs[g] = accs[g] + buf[t0 + j, pl.ds(coff + g * 16, 16)]
            return accs

        def reduce_doc(buf, d):
            off = d * L
            zero = jnp.zeros((16,), jnp.float32)

            def body(i, accs):
                return tuple(reduce_rows(buf, off, i * BLK, BLK, list(accs)))

            accs = lax.fori_loop(0, n_blk, body, (zero,) * n_groups)
            if tail:
                accs = reduce_rows(buf, off, n_blk * BLK, tail, list(accs))
            for g in range(n_groups):
                outblk[d, pl.ds(g * 16, 16)] = accs[g]

        gather_start(0, buf_a, sem_a)

        def pair_body(i, _):
            d0 = 2 * i
            gather_start(d0 + 1, buf_b, sem_b)
            gather_wait(buf_a, sem_a)
            reduce_doc(buf_a, d0)

            @pl.when(d0 + 2 < dpw)
            def _():
                gather_start(d0 + 2, buf_a, sem_a)

            gather_wait(buf_b, sem_b)
            reduce_doc(buf_b, d0 + 1)
            return 0

        lax.fori_loop(0, dpw // 2, pair_body, 0)

        pltpu.sync_copy(outblk, out_hbm.at[pl.ds(base_doc, dpw)])

    return sc_sum


def _linear_body(sums_ref, len_ref, w_ref, b_ref, out_ref):
    inv = 1.0 / jnp.maximum(len_ref[...], 1).astype(jnp.float32)
    emb = sums_ref[...] * inv
    out_ref[...] = (
        jnp.dot(emb, w_ref[...], preferred_element_type=jnp.float32) + b_ref[...]
    )


def _tc_linear(sums, lens2d, W, b2d, BLK=512):
    B, D = sums.shape
    NL = W.shape[1]
    return pl.pallas_call(
        _linear_body,
        grid=(B // BLK,),
        in_specs=[
            pl.BlockSpec((BLK, D), lambda i: (i, 0)),
            pl.BlockSpec((BLK, 1), lambda i: (i, 0)),
            pl.BlockSpec((D, NL), lambda i: (0, 0)),
            pl.BlockSpec((1, NL), lambda i: (0, 0)),
        ],
        out_specs=pl.BlockSpec((BLK, NL), lambda i: (i, 0)),
        out_shape=jax.ShapeDtypeStruct((B, NL), jnp.float32),
    )(sums, lens2d, W, b2d)


@jax.jit
def kernel(doc_token, doc_token_len, token_table, W, b):
    B, L = doc_token.shape
    V, D = token_table.shape
    NW = 32
    sc_sum = _make_sc_sum(B, L, V // 2, D, NW)
    tokens = doc_token.reshape(-1).astype(jnp.int32)
    idx2 = jax.lax.shift_right_logical(tokens, 1)
    table2 = token_table.reshape(V // 2, 2 * D)
    sums = sc_sum(tokens, idx2, table2)
    lens2d = doc_token_len.reshape(B, 1)
    b2d = b.reshape(1, -1)
    return _tc_linear(sums, lens2d, W, b2d)
